# Initial kernel scaffold; baseline (speedup 1.0000x reference)
import functools

import jax
import jax.numpy as jnp
from jax import lax
from jax.experimental import pallas as pl
from jax.experimental.pallas import tpu as pltpu
from jax.experimental.pallas import tpu_sc as plsc

N = 10000
E = 320000
D = 128
C = 64

NC = 2
NS = 16
NW = NC * NS
N_PAD = 10240
ROWS_PER_TILE = N_PAD // NS
K = 128
NCHUNKS = E // K

_mesh = plsc.VectorSubcoreMesh(core_axis_name="c", subcore_axis_name="s")


def _n_my_chunks(wid):
    return jnp.where(wid < NCHUNKS % NW, NCHUNKS // NW + 1, NCHUNKS // NW)


@functools.partial(
    pl.kernel,
    out_type=jax.ShapeDtypeStruct((NC, N_PAD), jnp.float32),
    mesh=_mesh,
    scratch_types=[
        pltpu.VMEM((K,), jnp.int32),
        pltpu.VMEM((K,), jnp.float32),
        pltpu.VMEM((ROWS_PER_TILE,), jnp.float32),
        pltpu.VMEM_SHARED((N_PAD,), jnp.float32),
    ],
)
def _deg_kernel(dst_hbm, out_hbm, didx, ones, zbuf, acc):
    cid = lax.axis_index("c")
    sid = lax.axis_index("s")
    wid = cid * NS + sid

    def fill(i, _):
        ones[pl.ds(i * 16, 16)] = jnp.full((16,), 1.0, jnp.float32)
        zbuf[pl.ds(i * 16, 16)] = jnp.zeros((16,), jnp.float32)
        return 0

    lax.fori_loop(0, ROWS_PER_TILE // 16, fill, 0)
    pltpu.sync_copy(zbuf, acc.at[pl.ds(sid * ROWS_PER_TILE, ROWS_PER_TILE)])
    plsc.subcore_barrier()

    def body(i, _):
        off = (wid + i * NW) * K
        pltpu.sync_copy(dst_hbm.at[pl.ds(off, K)], didx)
        pltpu.sync_copy(ones, acc.at[didx], add=True)
        return 0

    lax.fori_loop(0, _n_my_chunks(wid), body, 0)
    plsc.subcore_barrier()

    pltpu.sync_copy(
        acc.at[pl.ds(sid * ROWS_PER_TILE, ROWS_PER_TILE)],
        out_hbm.at[cid, pl.ds(sid * ROWS_PER_TILE, ROWS_PER_TILE)],
    )


@functools.partial(
    pl.kernel,
    out_type=jax.ShapeDtypeStruct((NC, N_PAD, D), jnp.float32),
    mesh=_mesh,
    scratch_types=[
        pltpu.VMEM((K,), jnp.int32),
        pltpu.VMEM((K,), jnp.int32),
        pltpu.VMEM((K, D), jnp.float32),
        pltpu.VMEM((ROWS_PER_TILE, D), jnp.float32),
        pltpu.VMEM_SHARED((N_PAD, D), jnp.float32),
    ],
)
def _seg_kernel(x_hbm, src_hbm, dst_hbm, out_hbm, sidx, didx, rows, zbuf, acc):
    cid = lax.axis_index("c")
    sid = lax.axis_index("s")
    wid = cid * NS + sid

    def fill(r, _):
        for c8 in range(D // 16):
            zbuf[r, pl.ds(c8 * 16, 16)] = jnp.zeros((16,), jnp.float32)
        return 0

    lax.fori_loop(0, ROWS_PER_TILE, fill, 0)
    pltpu.sync_copy(zbuf, acc.at[pl.ds(sid * ROWS_PER_TILE, ROWS_PER_TILE)])
    plsc.subcore_barrier()

    def body(i, _):
        off = (wid + i * NW) * K
        pltpu.sync_copy(src_hbm.at[pl.ds(off, K)], sidx)
        pltpu.sync_copy(dst_hbm.at[pl.ds(off, K)], didx)
        pltpu.sync_copy(x_hbm.at[sidx], rows)
        pltpu.sync_copy(rows, acc.at[didx], add=True)
        return 0

    lax.fori_loop(0, _n_my_chunks(wid), body, 0)
    plsc.subcore_barrier()

    pltpu.sync_copy(
        acc.at[pl.ds(sid * ROWS_PER_TILE, ROWS_PER_TILE)],
        out_hbm.at[cid, pl.ds(sid * ROWS_PER_TILE, ROWS_PER_TILE)],
    )


def _norm_scale_body(degp_ref, x_ref, a_ref, nrm_ref):
    deg = degp_ref[0] + degp_ref[1]
    nrm = lax.rsqrt(jnp.maximum(deg, 1.0))
    nrm_ref[...] = nrm
    a_ref[...] = x_ref[...] * nrm[:, None]


def _combine_scale_body(part_ref, nrm_ref, c_ref):
    nrm = nrm_ref[...]
    c_ref[...] = (part_ref[0] + part_ref[1]) * (nrm * nrm)[:, None]


def _final_body(part_ref, nrm_ref, w_ref, out_ref, h_ref):
    h2 = (part_ref[0] + part_ref[1]) * nrm_ref[...][:, None]
    h = jnp.dot(h2, w_ref[...], preferred_element_type=jnp.float32)
    h_ref[...] = h
    m = jnp.max(h, axis=1, keepdims=True)
    e = jnp.exp(h - m)
    out_ref[...] = e / jnp.sum(e, axis=1, keepdims=True)


def kernel(features, edge_index, lg, lg_x, W):
    del lg, lg_x
    src = edge_index[0]
    dst = edge_index[1]
    x = jnp.zeros((N_PAD, D), jnp.float32).at[:N].set(features)

    degp = _deg_kernel(dst)

    a, nrm = pl.pallas_call(
        _norm_scale_body,
        out_shape=(
            jax.ShapeDtypeStruct((N_PAD, D), jnp.float32),
            jax.ShapeDtypeStruct((N_PAD,), jnp.float32),
        ),
    )(degp, x)

    bp = _seg_kernel(a, src, dst)

    c = pl.pallas_call(
        _combine_scale_body,
        out_shape=jax.ShapeDtypeStruct((N_PAD, D), jnp.float32),
    )(bp, nrm)

    dp = _seg_kernel(c, src, dst)

    out, h = pl.pallas_call(
        _final_body,
        out_shape=(
            jax.ShapeDtypeStruct((N_PAD, C), jnp.float32),
            jax.ShapeDtypeStruct((N_PAD, C), jnp.float32),
        ),
    )(dp, nrm, W)

    return (out[:N], h[:N])


# speedup vs baseline: 5.7076x; 5.7076x over previous
import functools

import jax
import jax.numpy as jnp
from jax import lax
from jax.experimental import pallas as pl
from jax.experimental.pallas import tpu as pltpu
from jax.experimental.pallas import tpu_sc as plsc

N = 10000
E = 320000
D = 128
C = 64

NC = 2
NS = 16
NW = NC * NS
N_PAD = 10240
ROWS_PER_TILE = N_PAD // NS
K = 128
NCHUNKS = E // K

@functools.cache
def _mesh():
    return plsc.VectorSubcoreMesh(
        core_axis_name="c", subcore_axis_name="s", num_cores=NC, num_subcores=NS
    )


def _n_my_chunks(wid):
    return jnp.where(wid < NCHUNKS % NW, NCHUNKS // NW + 1, NCHUNKS // NW)


def _deg_body(dst_hbm, out_hbm, didx, ones, zbuf, acc):
    cid = lax.axis_index("c")
    sid = lax.axis_index("s")
    wid = cid * NS + sid

    def fill_ones(i, _):
        ones[pl.ds(i * 16, 16)] = jnp.full((16,), 1.0, jnp.float32)
        return 0

    def fill_zero(i, _):
        zbuf[pl.ds(i * 16, 16)] = jnp.zeros((16,), jnp.float32)
        return 0

    lax.fori_loop(0, K // 16, fill_ones, 0)
    lax.fori_loop(0, ROWS_PER_TILE // 16, fill_zero, 0)
    pltpu.sync_copy(zbuf, acc.at[pl.ds(sid * ROWS_PER_TILE, ROWS_PER_TILE)])
    plsc.subcore_barrier()

    def body(i, _):
        off = (wid + i * NW) * K
        pltpu.sync_copy(dst_hbm.at[pl.ds(off, K)], didx)
        pltpu.sync_copy(ones, acc.at[didx], add=True)
        return 0

    lax.fori_loop(0, _n_my_chunks(wid), body, 0)
    plsc.subcore_barrier()

    pltpu.sync_copy(
        acc.at[pl.ds(sid * ROWS_PER_TILE, ROWS_PER_TILE)],
        out_hbm.at[cid, pl.ds(sid * ROWS_PER_TILE, ROWS_PER_TILE)],
    )


def _seg_body(x_hbm, src_hbm, dst_hbm, out_hbm, sidx, didx, rows, zbuf, acc):
    cid = lax.axis_index("c")
    sid = lax.axis_index("s")
    wid = cid * NS + sid

    def fill(r, _):
        for c8 in range(D // 16):
            zbuf[r, pl.ds(c8 * 16, 16)] = jnp.zeros((16,), jnp.float32)
        return 0

    lax.fori_loop(0, 64, fill, 0)

    def zero_out(t, _):
        pltpu.sync_copy(zbuf, acc.at[pl.ds(sid * ROWS_PER_TILE + t * 64, 64)])
        return 0

    lax.fori_loop(0, ROWS_PER_TILE // 64, zero_out, 0)
    plsc.subcore_barrier()

    def body(i, _):
        off = (wid + i * NW) * K
        pltpu.sync_copy(src_hbm.at[pl.ds(off, K)], sidx)
        pltpu.sync_copy(dst_hbm.at[pl.ds(off, K)], didx)
        pltpu.sync_copy(x_hbm.at[sidx], rows)
        pltpu.sync_copy(rows, acc.at[didx], add=True)
        return 0

    lax.fori_loop(0, _n_my_chunks(wid), body, 0)
    plsc.subcore_barrier()

    pltpu.sync_copy(
        acc.at[pl.ds(sid * ROWS_PER_TILE, ROWS_PER_TILE)],
        out_hbm.at[cid, pl.ds(sid * ROWS_PER_TILE, ROWS_PER_TILE)],
    )


@functools.cache
def _deg_kernel():
    return pl.kernel(
        _deg_body,
        out_type=jax.ShapeDtypeStruct((NC, N_PAD), jnp.float32),
        mesh=_mesh(),
        scratch_types=[
            pltpu.VMEM((K,), jnp.int32),
            pltpu.VMEM((K,), jnp.float32),
            pltpu.VMEM((ROWS_PER_TILE,), jnp.float32),
            pltpu.VMEM_SHARED((N_PAD,), jnp.float32),
        ],
    )


@functools.cache
def _seg_kernel():
    return pl.kernel(
        _seg_body,
        out_type=jax.ShapeDtypeStruct((NC, N_PAD, D), jnp.float32),
        mesh=_mesh(),
        scratch_types=[
            pltpu.VMEM((K,), jnp.int32),
            pltpu.VMEM((K,), jnp.int32),
            pltpu.VMEM((K, D), jnp.float32),
            pltpu.VMEM((64, D), jnp.float32),
            pltpu.VMEM_SHARED((N_PAD, D), jnp.float32),
        ],
    )


def _norm_scale_body(degp_ref, x_ref, a_ref, nrm_ref):
    deg = degp_ref[0] + degp_ref[1]
    nrm = lax.rsqrt(jnp.maximum(deg, 1.0))
    nrm_ref[...] = nrm
    a_ref[...] = x_ref[...] * nrm[:, None]


def _combine_scale_body(part_ref, nrm_ref, c_ref):
    nrm = nrm_ref[...]
    c_ref[...] = (part_ref[0] + part_ref[1]) * (nrm * nrm)[:, None]


def _final_body(part_ref, nrm_ref, w_ref, out_ref, h_ref):
    h2 = (part_ref[0] + part_ref[1]) * nrm_ref[...][:, None]
    h = jnp.dot(h2, w_ref[...], preferred_element_type=jnp.float32)
    h_ref[...] = h
    m = jnp.max(h, axis=1, keepdims=True)
    e = jnp.exp(h - m)
    out_ref[...] = e / jnp.sum(e, axis=1, keepdims=True)


def kernel(features, edge_index, lg, lg_x, W):
    del lg, lg_x
    src = edge_index[0]
    dst = edge_index[1]
    x = jnp.zeros((N_PAD, D), jnp.float32).at[:N].set(features)

    degp = _deg_kernel()(dst)

    a, nrm = pl.pallas_call(
        _norm_scale_body,
        out_shape=(
            jax.ShapeDtypeStruct((N_PAD, D), jnp.float32),
            jax.ShapeDtypeStruct((N_PAD,), jnp.float32),
        ),
    )(degp, x)

    bp = _seg_kernel()(a, src, dst)

    c = pl.pallas_call(
        _combine_scale_body,
        out_shape=jax.ShapeDtypeStruct((N_PAD, D), jnp.float32),
    )(bp, nrm)

    dp = _seg_kernel()(c, src, dst)

    out, h = pl.pallas_call(
        _final_body,
        out_shape=(
            jax.ShapeDtypeStruct((N_PAD, C), jnp.float32),
            jax.ShapeDtypeStruct((N_PAD, C), jnp.float32),
        ),
    )(dp, nrm, W)

    return (out[:N], h[:N])


# speedup vs baseline: 9.3208x; 1.6331x over previous
import functools

import jax
import jax.numpy as jnp
from jax import lax
from jax.experimental import pallas as pl
from jax.experimental.pallas import tpu as pltpu
from jax.experimental.pallas import tpu_sc as plsc

N = 10000
E = 320000
D = 128
C = 64

NC = 2
NS = 16
NW = NC * NS
N_PAD = 10240
ROWS_PER_TILE = N_PAD // NS
K = 128
NCHUNKS = E // K

@functools.cache
def _mesh():
    return plsc.VectorSubcoreMesh(
        core_axis_name="c", subcore_axis_name="s", num_cores=NC, num_subcores=NS
    )


def _n_my_chunks(wid):
    return jnp.where(wid < NCHUNKS % NW, NCHUNKS // NW + 1, NCHUNKS // NW)


def _deg_body(dst_hbm, out_hbm, didx, ones, zbuf, acc):
    cid = lax.axis_index("c")
    sid = lax.axis_index("s")
    wid = cid * NS + sid

    def fill_ones(i, _):
        ones[pl.ds(i * 16, 16)] = jnp.full((16,), 1.0, jnp.float32)
        return 0

    def fill_zero(i, _):
        zbuf[pl.ds(i * 16, 16)] = jnp.zeros((16,), jnp.float32)
        return 0

    lax.fori_loop(0, K // 16, fill_ones, 0)
    lax.fori_loop(0, ROWS_PER_TILE // 16, fill_zero, 0)
    pltpu.sync_copy(zbuf, acc.at[pl.ds(sid * ROWS_PER_TILE, ROWS_PER_TILE)])
    plsc.subcore_barrier()

    def body(i, _):
        off = (wid + i * NW) * K
        pltpu.sync_copy(dst_hbm.at[pl.ds(off, K)], didx)
        pltpu.sync_copy(ones, acc.at[didx], add=True)
        return 0

    lax.fori_loop(0, _n_my_chunks(wid), body, 0)
    plsc.subcore_barrier()

    pltpu.sync_copy(
        acc.at[pl.ds(sid * ROWS_PER_TILE, ROWS_PER_TILE)],
        out_hbm.at[cid, pl.ds(sid * ROWS_PER_TILE, ROWS_PER_TILE)],
    )


def _seg_body(
    x_hbm, src_hbm, dst_hbm, out_hbm,
    sidx, didx, rows, zbuf, acc, sem_si, sem_di, sem_g, sem_s,
):
    cid = lax.axis_index("c")
    sid = lax.axis_index("s")
    wid = cid * NS + sid
    nch = _n_my_chunks(wid)

    def fill(r, _):
        for c8 in range(D // 16):
            zbuf[r, pl.ds(c8 * 16, 16)] = jnp.zeros((16,), jnp.float32)
        return 0

    lax.fori_loop(0, 64, fill, 0)

    def zero_out(t, _):
        pltpu.sync_copy(zbuf, acc.at[pl.ds(sid * ROWS_PER_TILE + t * 64, 64)])
        return 0

    lax.fori_loop(0, ROWS_PER_TILE // 64, zero_out, 0)
    plsc.subcore_barrier()

    def idx_start(i, b):
        off = (wid + i * NW) * K
        pltpu.async_copy(src_hbm.at[pl.ds(off, K)], sidx.at[b], sem_si.at[b])
        pltpu.async_copy(dst_hbm.at[pl.ds(off, K)], didx.at[b], sem_di.at[b])

    idx_start(0, 0)

    def body(i, _):
        b = lax.rem(i, 2)
        nb = 1 - b
        off = (wid + i * NW) * K
        pltpu.make_async_copy(src_hbm.at[pl.ds(off, K)], sidx.at[b], sem_si.at[b]).wait()
        pltpu.make_async_copy(dst_hbm.at[pl.ds(off, K)], didx.at[b], sem_di.at[b]).wait()
        pltpu.async_copy(x_hbm.at[sidx.at[b]], rows.at[b], sem_g.at[b])
        @pl.when(i >= 1)
        def _():
            pltpu.make_async_copy(rows.at[nb], acc.at[didx.at[nb]], sem_s.at[nb]).wait()

        @pl.when(i + 1 < nch)
        def _():
            idx_start(i + 1, nb)

        pltpu.make_async_copy(x_hbm.at[sidx.at[b]], rows.at[b], sem_g.at[b]).wait()
        pltpu.async_copy(rows.at[b], acc.at[didx.at[b]], sem_s.at[b], add=True)
        return 0

    lax.fori_loop(0, nch, body, 0)
    bl = lax.rem(nch - 1, 2)
    pltpu.make_async_copy(rows.at[bl], acc.at[didx.at[bl]], sem_s.at[bl]).wait()
    plsc.subcore_barrier()

    pltpu.sync_copy(
        acc.at[pl.ds(sid * ROWS_PER_TILE, ROWS_PER_TILE)],
        out_hbm.at[cid, pl.ds(sid * ROWS_PER_TILE, ROWS_PER_TILE)],
    )


@functools.cache
def _deg_kernel():
    return pl.kernel(
        _deg_body,
        out_type=jax.ShapeDtypeStruct((NC, N_PAD), jnp.float32),
        mesh=_mesh(),
        scratch_types=[
            pltpu.VMEM((K,), jnp.int32),
            pltpu.VMEM((K,), jnp.float32),
            pltpu.VMEM((ROWS_PER_TILE,), jnp.float32),
            pltpu.VMEM_SHARED((N_PAD,), jnp.float32),
        ],
    )


@functools.cache
def _seg_kernel():
    return pl.kernel(
        _seg_body,
        out_type=jax.ShapeDtypeStruct((NC, N_PAD, D), jnp.float32),
        mesh=_mesh(),
        scratch_types=[
            pltpu.VMEM((2, K), jnp.int32),
            pltpu.VMEM((2, K), jnp.int32),
            pltpu.VMEM((2, K, D), jnp.float32),
            pltpu.VMEM((64, D), jnp.float32),
            pltpu.VMEM_SHARED((N_PAD, D), jnp.float32),
            pltpu.SemaphoreType.DMA((2,)),
            pltpu.SemaphoreType.DMA((2,)),
            pltpu.SemaphoreType.DMA((2,)),
            pltpu.SemaphoreType.DMA((2,)),
        ],
    )


def _norm_scale_body(degp_ref, x_ref, a_ref, nrm_ref):
    deg = degp_ref[0] + degp_ref[1]
    nrm = lax.rsqrt(jnp.maximum(deg, 1.0))
    nrm_ref[...] = nrm
    a_ref[...] = x_ref[...] * nrm[:, None]


def _combine_scale_body(part_ref, nrm_ref, c_ref):
    nrm = nrm_ref[...]
    c_ref[...] = (part_ref[0] + part_ref[1]) * (nrm * nrm)[:, None]


def _final_body(part_ref, nrm_ref, w_ref, out_ref, h_ref):
    h2 = (part_ref[0] + part_ref[1]) * nrm_ref[...][:, None]
    h = jnp.dot(h2, w_ref[...], preferred_element_type=jnp.float32)
    h_ref[...] = h
    m = jnp.max(h, axis=1, keepdims=True)
    e = jnp.exp(h - m)
    out_ref[...] = e / jnp.sum(e, axis=1, keepdims=True)


def kernel(features, edge_index, lg, lg_x, W):
    del lg, lg_x
    src = edge_index[0]
    dst = edge_index[1]
    x = jnp.zeros((N_PAD, D), jnp.float32).at[:N].set(features)

    degp = _deg_kernel()(dst)

    a, nrm = pl.pallas_call(
        _norm_scale_body,
        out_shape=(
            jax.ShapeDtypeStruct((N_PAD, D), jnp.float32),
            jax.ShapeDtypeStruct((N_PAD,), jnp.float32),
        ),
    )(degp, x)

    bp = _seg_kernel()(a, src, dst)

    c = pl.pallas_call(
        _combine_scale_body,
        out_shape=jax.ShapeDtypeStruct((N_PAD, D), jnp.float32),
    )(bp, nrm)

    dp = _seg_kernel()(c, src, dst)

    out, h = pl.pallas_call(
        _final_body,
        out_shape=(
            jax.ShapeDtypeStruct((N_PAD, C), jnp.float32),
            jax.ShapeDtypeStruct((N_PAD, C), jnp.float32),
        ),
    )(dp, nrm, W)

    return (out[:N], h[:N])
